# baseline (speedup 1.0000x reference)
import functools

import jax
import jax.numpy as jnp
from jax import lax
from jax.experimental import pallas as pl
from jax.experimental.pallas import tpu as pltpu
from jax.experimental.pallas import tpu_sc as plsc

N = 10000
D = 128
EPS = 1e-5

NC = 2
NS = 16
NW = NC * NS
BATCH = 128

ACC_PER_TILE = 632
ACC_ROWS = NS * ACC_PER_TILE
DEG_PER_TILE = 640
DEG_ROWS = NS * DEG_PER_TILE
PAD_DST = 10008

_mesh = plsc.VectorSubcoreMesh(core_axis_name="c", subcore_axis_name="s")


CH = 40
NPAIR = CH // 2
NB0 = 80
NB1 = 80
TB = NS * (NB0 + NB1)


def _sc_body(with_deg, h_hbm, src_hbm, dst_hbm, *rest):
    if with_deg:
        (acc_out, deg_out, src_v, dst_v, rows_a, rows_b, ones_v,
         acc_sh, deg_sh, semA0, semA1, semB0, semB1, semD) = rest
    else:
        (acc_out, src_v, dst_v, rows_a, rows_b,
         acc_sh, semA0, semA1, semB0, semB1) = rest
    c = lax.axis_index("c")
    s = lax.axis_index("s")

    zeros16 = jnp.zeros((16,), jnp.float32)
    ones16 = jnp.ones((16,), jnp.float32)

    def _zero_row_body(i, carry):
        for k in range(D // 16):
            rows_a[i, pl.ds(k * 16, 16)] = zeros16
        return carry

    lax.fori_loop(0, BATCH, _zero_row_body, 0)
    if with_deg:
        for k in range(BATCH // 16):
            ones_v[pl.ds(k * 16, 16)] = ones16

    base = s * ACC_PER_TILE
    n_full = ACC_PER_TILE // BATCH
    rem = ACC_PER_TILE - n_full * BATCH
    for k in range(n_full):
        pltpu.sync_copy(rows_a, acc_sh.at[pl.ds(base + k * BATCH, BATCH)])
    if rem:
        pltpu.sync_copy(rows_a.at[pl.ds(0, rem)],
                        acc_sh.at[pl.ds(base + n_full * BATCH, rem)])
    if with_deg:
        dbase = s * DEG_PER_TILE
        for k in range(DEG_PER_TILE // BATCH):
            pltpu.sync_copy(rows_a.at[0], deg_sh.at[pl.ds(dbase + k * BATCH, BATCH)])
    plsc.subcore_barrier()

    nchunks = jnp.where(c == 0, NB0 // CH, NB1 // CH)
    base_b = jnp.where(c == 0, s * NB0, NS * NB0 + s * NB1)

    def _chunk_body(kc, carry):
        c0 = pl.multiple_of(base_b + kc * CH, 8)
        pltpu.sync_copy(src_hbm.at[pl.ds(c0, CH)], src_v)
        pltpu.sync_copy(dst_hbm.at[pl.ds(c0, CH)], dst_v)
        pltpu.async_copy(h_hbm.at[src_v.at[0]], rows_a, semA0)

        def _pair_body(p, carry):
            j0 = 2 * p
            j1 = j0 + 1

            pltpu.make_async_copy(h_hbm.at[src_v.at[j0]], rows_a, semA0).wait()
            pltpu.async_copy(rows_a, acc_sh.at[dst_v.at[j0]], semB0, add=True)
            if with_deg:
                pltpu.async_copy(ones_v, deg_sh.at[dst_v.at[j0]], semD, add=True)

            @pl.when(p > 0)
            def _():
                pltpu.make_async_copy(
                    rows_b, acc_sh.at[dst_v.at[j0 - 1]], semB1).wait()

            pltpu.async_copy(h_hbm.at[src_v.at[j1]], rows_b, semA1)

            pltpu.make_async_copy(h_hbm.at[src_v.at[j1]], rows_b, semA1).wait()
            pltpu.async_copy(rows_b, acc_sh.at[dst_v.at[j1]], semB1, add=True)
            if with_deg:
                pltpu.async_copy(ones_v, deg_sh.at[dst_v.at[j1]], semD, add=True)
            pltpu.make_async_copy(rows_a, acc_sh.at[dst_v.at[j0]], semB0).wait()

            @pl.when(p + 1 < NPAIR)
            def _():
                pltpu.async_copy(h_hbm.at[src_v.at[j0 + 2]], rows_a, semA0)

            return carry

        lax.fori_loop(0, NPAIR, _pair_body, 0)
        pltpu.make_async_copy(rows_b, acc_sh.at[dst_v.at[CH - 1]], semB1).wait()
        if with_deg:
            def _deg_drain(p, carry2):
                pltpu.make_async_copy(ones_v, deg_sh.at[dst_v.at[0]], semD).wait()
                return carry2

            lax.fori_loop(0, CH, _deg_drain, 0)
        return carry

    lax.fori_loop(0, nchunks, _chunk_body, 0)
    plsc.subcore_barrier()

    pltpu.sync_copy(acc_sh.at[pl.ds(base, ACC_PER_TILE)],
                    acc_out.at[c].at[pl.ds(base, ACC_PER_TILE)])
    if with_deg:
        pltpu.sync_copy(deg_sh.at[pl.ds(dbase, DEG_PER_TILE)],
                        deg_out.at[c].at[pl.ds(dbase, DEG_PER_TILE)])


def _make_sc_segment_sum(with_deg):
    out_type = [jax.ShapeDtypeStruct((NC, ACC_ROWS, D), jnp.float32)]
    scratch = [
        pltpu.VMEM((CH, BATCH), jnp.int32),
        pltpu.VMEM((CH, BATCH), jnp.int32),
        pltpu.VMEM((BATCH, D), jnp.float32),
        pltpu.VMEM((BATCH, D), jnp.float32),
    ]
    if with_deg:
        out_type.append(jax.ShapeDtypeStruct((NC, DEG_ROWS), jnp.float32))
        scratch.append(pltpu.VMEM((BATCH,), jnp.float32))
    scratch.append(pltpu.VMEM_SHARED((ACC_ROWS, D), jnp.float32))
    if with_deg:
        scratch.append(pltpu.VMEM_SHARED((DEG_ROWS,), jnp.float32))
    scratch += [pltpu.SemaphoreType.DMA] * (5 if with_deg else 4)
    return functools.partial(
        pl.kernel,
        mesh=_mesh,
        out_type=out_type,
        scratch_types=scratch,
    )(functools.partial(_sc_body, with_deg))


def _dense_body(accA_ref, accB_ref, degA_ref, degB_ref, h_ref,
                Wl_ref, Wr_ref, b_ref, z_ref, stats_ref):
    i = pl.program_id(0)
    deg = degA_ref[...] + degB_ref[...]
    inv = 1.0 / jnp.maximum(deg, 1.0)
    S = (accA_ref[...] + accB_ref[...]) * inv
    z = lax.dot_general(S, Wl_ref[...], (((1,), (1,)), ((), ())),
                        preferred_element_type=jnp.float32)
    z = z + lax.dot_general(h_ref[...], Wr_ref[...], (((1,), (1,)), ((), ())),
                            preferred_element_type=jnp.float32)
    z = z + b_ref[...]
    z_ref[...] = z
    s0 = jnp.sum(z, axis=0, keepdims=True)
    s1 = jnp.sum(z * z, axis=0, keepdims=True)
    upd = jnp.concatenate([s0, s1, jnp.zeros((6, D), jnp.float32)], axis=0)

    @pl.when(i == 0)
    def _():
        stats_ref[...] = jnp.zeros((8, D), jnp.float32)

    stats_ref[...] += upd


def _bn_body(z_ref, stats_ref, gamma_ref, beta_ref, out_ref):
    st = stats_ref[...]
    mu = st[0:1, :] * (1.0 / N)
    var = st[1:2, :] * (1.0 / N) - mu * mu
    a = gamma_ref[...] * lax.rsqrt(var + EPS)
    out_ref[...] = jnp.maximum((z_ref[...] - mu) * a + beta_ref[...], 0.0)


def _dense_bn_relu(accA, accB, degA, degB, h, Wl, Wr, b, gamma, beta):
    R = 1000
    G = N // R
    row = lambda i: (i, 0)
    const = lambda i: (0, 0)
    z, stats = pl.pallas_call(
        _dense_body,
        grid=(G,),
        in_specs=[
            pl.BlockSpec((R, D), row),
            pl.BlockSpec((R, D), row),
            pl.BlockSpec((R, 1), row),
            pl.BlockSpec((R, 1), row),
            pl.BlockSpec((R, D), row),
            pl.BlockSpec((D, D), const),
            pl.BlockSpec((D, D), const),
            pl.BlockSpec((1, D), const),
        ],
        out_specs=[
            pl.BlockSpec((R, D), row),
            pl.BlockSpec((8, D), const),
        ],
        out_shape=[
            jax.ShapeDtypeStruct((N, D), jnp.float32),
            jax.ShapeDtypeStruct((8, D), jnp.float32),
        ],
    )(accA, accB, degA, degB, h, Wl, Wr, b)
    return pl.pallas_call(
        _bn_body,
        grid=(G,),
        in_specs=[
            pl.BlockSpec((R, D), row),
            pl.BlockSpec((8, D), const),
            pl.BlockSpec((1, D), const),
            pl.BlockSpec((1, D), const),
        ],
        out_specs=pl.BlockSpec((R, D), row),
        out_shape=jax.ShapeDtypeStruct((N, D), jnp.float32),
    )(z, stats, gamma, beta)


def kernel(x, edge_index, W1l, W1r, b1, gamma1, beta1, W2l, W2r, b2, gamma2, beta2):
    E = edge_index.shape[1]
    epad = TB * BATCH
    npad = epad - E
    pad_src = (jnp.arange(npad, dtype=jnp.int32) * 97) % N
    pad_dst = N + (jnp.arange(npad, dtype=jnp.int32) % (ACC_ROWS - N))
    src = jnp.concatenate([edge_index[0], pad_src]).reshape(TB, BATCH)
    dst = jnp.concatenate(
        [edge_index[1], pad_dst.astype(jnp.int32)]).reshape(TB, BATCH)

    seg_deg = _make_sc_segment_sum(True)
    seg = _make_sc_segment_sum(False)

    def dense(acc, deg, h, Wl, Wr, b, gamma, beta):
        return _dense_bn_relu(
            acc[0, :N], acc[1, :N],
            deg[0, :N, None], deg[1, :N, None],
            h, Wl, Wr, b.reshape(1, D), gamma.reshape(1, D), beta.reshape(1, D))

    acc1, deg = seg_deg(x, src, dst)
    h1 = dense(acc1, deg, x, W1l, W1r, b1, gamma1, beta1)
    acc2, = seg(h1, src, dst)
    return dense(acc2, deg, h1, W2l, W2r, b2, gamma2, beta2)


# speedup vs baseline: 1.1533x; 1.1533x over previous
import functools

import jax
import jax.numpy as jnp
from jax import lax
from jax.experimental import pallas as pl
from jax.experimental.pallas import tpu as pltpu
from jax.experimental.pallas import tpu_sc as plsc

N = 10000
D = 128
EPS = 1e-5

NC = 2
NS = 16
NW = NC * NS
BATCH = 128

ACC_PER_TILE = 632
ACC_ROWS = NS * ACC_PER_TILE
DEG_PER_TILE = 640
DEG_ROWS = NS * DEG_PER_TILE
PAD_DST = 10008

_mesh = plsc.VectorSubcoreMesh(core_axis_name="c", subcore_axis_name="s")


CH = 40
NPAIR = CH // 2
NB0 = 80
NB1 = 80
TB = NS * (NB0 + NB1)


def _sc_body(with_deg, h_hbm, src_hbm, dst_hbm, *rest):
    if with_deg:
        (acc_out, deg_out, src_v, dst_v, rows_a, rows_b, ones_v,
         acc_sh, deg_sh, semA0, semA1, semB0, semB1, semD) = rest
    else:
        (acc_out, src_v, dst_v, rows_a, rows_b,
         acc_sh, semA0, semA1, semB0, semB1) = rest
    c = lax.axis_index("c")
    s = lax.axis_index("s")

    zeros16 = jnp.zeros((16,), jnp.float32)
    ones16 = jnp.ones((16,), jnp.float32)

    def _zero_row_body(i, carry):
        for k in range(D // 16):
            rows_a[i, pl.ds(k * 16, 16)] = zeros16
        return carry

    lax.fori_loop(0, BATCH, _zero_row_body, 0)
    if with_deg:
        for k in range(BATCH // 16):
            ones_v[pl.ds(k * 16, 16)] = ones16

    base = s * ACC_PER_TILE
    n_full = ACC_PER_TILE // BATCH
    rem = ACC_PER_TILE - n_full * BATCH
    for k in range(n_full):
        pltpu.sync_copy(rows_a, acc_sh.at[pl.ds(base + k * BATCH, BATCH)])
    if rem:
        pltpu.sync_copy(rows_a.at[pl.ds(0, rem)],
                        acc_sh.at[pl.ds(base + n_full * BATCH, rem)])
    if with_deg:
        dbase = s * DEG_PER_TILE
        for k in range(DEG_PER_TILE // BATCH):
            pltpu.sync_copy(rows_a.at[0], deg_sh.at[pl.ds(dbase + k * BATCH, BATCH)])
    plsc.subcore_barrier()

    nchunks = jnp.where(c == 0, NB0 // CH, NB1 // CH)
    base_b = jnp.where(c == 0, s * NB0, NS * NB0 + s * NB1)

    def _chunk_body(kc, carry):
        c0 = pl.multiple_of(base_b + kc * CH, 8)
        pltpu.sync_copy(src_hbm.at[pl.ds(c0, CH)], src_v)
        pltpu.sync_copy(dst_hbm.at[pl.ds(c0, CH)], dst_v)
        pltpu.async_copy(h_hbm.at[src_v.at[0]], rows_a, semA0)

        def _pair_body(p, carry):
            j0 = 2 * p
            j1 = j0 + 1

            @pl.when(p > 0)
            def _():
                pltpu.make_async_copy(
                    rows_b, acc_sh.at[dst_v.at[j0 - 1]], semB1).wait()

            pltpu.async_copy(h_hbm.at[src_v.at[j1]], rows_b, semA1)
            pltpu.make_async_copy(h_hbm.at[src_v.at[j0]], rows_a, semA0).wait()
            pltpu.async_copy(rows_a, acc_sh.at[dst_v.at[j0]], semB0, add=True)
            if with_deg:
                pltpu.async_copy(ones_v, deg_sh.at[dst_v.at[j0]], semD, add=True)
            pltpu.make_async_copy(h_hbm.at[src_v.at[j1]], rows_b, semA1).wait()
            pltpu.make_async_copy(rows_a, acc_sh.at[dst_v.at[j0]], semB0).wait()

            @pl.when(p + 1 < NPAIR)
            def _():
                pltpu.async_copy(h_hbm.at[src_v.at[j0 + 2]], rows_a, semA0)

            pltpu.async_copy(rows_b, acc_sh.at[dst_v.at[j1]], semB1, add=True)
            if with_deg:
                pltpu.async_copy(ones_v, deg_sh.at[dst_v.at[j1]], semD, add=True)
            return carry

        lax.fori_loop(0, NPAIR, _pair_body, 0)
        pltpu.make_async_copy(rows_b, acc_sh.at[dst_v.at[CH - 1]], semB1).wait()
        if with_deg:
            def _deg_drain(p, carry2):
                pltpu.make_async_copy(ones_v, deg_sh.at[dst_v.at[0]], semD).wait()
                return carry2

            lax.fori_loop(0, CH, _deg_drain, 0)
        return carry

    lax.fori_loop(0, nchunks, _chunk_body, 0)
    plsc.subcore_barrier()

    pltpu.sync_copy(acc_sh.at[pl.ds(base, ACC_PER_TILE)],
                    acc_out.at[c].at[pl.ds(base, ACC_PER_TILE)])
    if with_deg:
        pltpu.sync_copy(deg_sh.at[pl.ds(dbase, DEG_PER_TILE)],
                        deg_out.at[c].at[pl.ds(dbase, DEG_PER_TILE)])


def _make_sc_segment_sum(with_deg):
    out_type = [jax.ShapeDtypeStruct((NC, ACC_ROWS, D), jnp.float32)]
    scratch = [
        pltpu.VMEM((CH, BATCH), jnp.int32),
        pltpu.VMEM((CH, BATCH), jnp.int32),
        pltpu.VMEM((BATCH, D), jnp.float32),
        pltpu.VMEM((BATCH, D), jnp.float32),
    ]
    if with_deg:
        out_type.append(jax.ShapeDtypeStruct((NC, DEG_ROWS), jnp.float32))
        scratch.append(pltpu.VMEM((BATCH,), jnp.float32))
    scratch.append(pltpu.VMEM_SHARED((ACC_ROWS, D), jnp.float32))
    if with_deg:
        scratch.append(pltpu.VMEM_SHARED((DEG_ROWS,), jnp.float32))
    scratch += [pltpu.SemaphoreType.DMA] * (5 if with_deg else 4)
    return functools.partial(
        pl.kernel,
        mesh=_mesh,
        out_type=out_type,
        scratch_types=scratch,
    )(functools.partial(_sc_body, with_deg))


def _dense_body(accA_ref, accB_ref, degA_ref, degB_ref, h_ref,
                Wl_ref, Wr_ref, b_ref, z_ref, stats_ref):
    i = pl.program_id(0)
    deg = degA_ref[...] + degB_ref[...]
    inv = 1.0 / jnp.maximum(deg, 1.0)
    S = (accA_ref[0] + accB_ref[0]) * inv
    z = lax.dot_general(S, Wl_ref[...], (((1,), (1,)), ((), ())),
                        preferred_element_type=jnp.float32)
    z = z + lax.dot_general(h_ref[...], Wr_ref[...], (((1,), (1,)), ((), ())),
                            preferred_element_type=jnp.float32)
    z = z + b_ref[...]
    z_ref[...] = z
    s0 = jnp.sum(z, axis=0, keepdims=True)
    s1 = jnp.sum(z * z, axis=0, keepdims=True)
    upd = jnp.concatenate([s0, s1, jnp.zeros((6, D), jnp.float32)], axis=0)

    @pl.when(i == 0)
    def _():
        stats_ref[...] = jnp.zeros((8, D), jnp.float32)

    stats_ref[...] += upd


def _bn_body(z_ref, stats_ref, gamma_ref, beta_ref, out_ref):
    st = stats_ref[...]
    mu = st[0:1, :] * (1.0 / N)
    var = st[1:2, :] * (1.0 / N) - mu * mu
    a = gamma_ref[...] * lax.rsqrt(var + EPS)
    out_ref[...] = jnp.maximum((z_ref[...] - mu) * a + beta_ref[...], 0.0)


def _dense_bn_relu(acc3d, degA, degB, h, Wl, Wr, b, gamma, beta):
    R = 1000
    G = N // R
    row = lambda i: (i, 0)
    const = lambda i: (0, 0)
    z, stats = pl.pallas_call(
        _dense_body,
        grid=(G,),
        in_specs=[
            pl.BlockSpec((1, R, D), lambda i: (0, i, 0)),
            pl.BlockSpec((1, R, D), lambda i: (1, i, 0)),
            pl.BlockSpec((R, 1), row),
            pl.BlockSpec((R, 1), row),
            pl.BlockSpec((R, D), row),
            pl.BlockSpec((D, D), const),
            pl.BlockSpec((D, D), const),
            pl.BlockSpec((1, D), const),
        ],
        out_specs=[
            pl.BlockSpec((R, D), row),
            pl.BlockSpec((8, D), const),
        ],
        out_shape=[
            jax.ShapeDtypeStruct((N, D), jnp.float32),
            jax.ShapeDtypeStruct((8, D), jnp.float32),
        ],
    )(acc3d, acc3d, degA, degB, h, Wl, Wr, b)
    return pl.pallas_call(
        _bn_body,
        grid=(G,),
        in_specs=[
            pl.BlockSpec((R, D), row),
            pl.BlockSpec((8, D), const),
            pl.BlockSpec((1, D), const),
            pl.BlockSpec((1, D), const),
        ],
        out_specs=pl.BlockSpec((R, D), row),
        out_shape=jax.ShapeDtypeStruct((N, D), jnp.float32),
    )(z, stats, gamma, beta)


def kernel(x, edge_index, W1l, W1r, b1, gamma1, beta1, W2l, W2r, b2, gamma2, beta2):
    E = edge_index.shape[1]
    epad = TB * BATCH
    npad = epad - E
    pad_src = (jnp.arange(npad, dtype=jnp.int32) * 97) % N
    pad_dst = N + (jnp.arange(npad, dtype=jnp.int32) % (ACC_ROWS - N))
    src = jnp.concatenate([edge_index[0], pad_src]).reshape(TB, BATCH)
    dst = jnp.concatenate(
        [edge_index[1], pad_dst.astype(jnp.int32)]).reshape(TB, BATCH)

    seg_deg = _make_sc_segment_sum(True)
    seg = _make_sc_segment_sum(False)

    def dense(acc, deg, h, Wl, Wr, b, gamma, beta):
        return _dense_bn_relu(
            acc,
            deg[0, :N, None], deg[1, :N, None],
            h, Wl, Wr, b.reshape(1, D), gamma.reshape(1, D), beta.reshape(1, D))

    acc1, deg = seg_deg(x, src, dst)
    h1 = dense(acc1, deg, x, W1l, W1r, b1, gamma1, beta1)
    acc2, = seg(h1, src, dst)
    return dense(acc2, deg, h1, W2l, W2r, b2, gamma2, beta2)


# speedup vs baseline: 1.2638x; 1.0958x over previous
import functools

import jax
import jax.numpy as jnp
import numpy as np
from jax import lax
from jax.experimental import pallas as pl
from jax.experimental.pallas import tpu as pltpu
from jax.experimental.pallas import tpu_sc as plsc

N = 10000
D = 128
EPS = 1e-5

NC = 2
NS = 16
NW = NC * NS
BATCH = 128

ACC_PER_TILE = 632
ACC_ROWS = NS * ACC_PER_TILE
DEG_PER_TILE = 640
DEG_ROWS = NS * DEG_PER_TILE

_mesh = plsc.VectorSubcoreMesh(core_axis_name="c", subcore_axis_name="s")


CH = 40
NPAIR = CH // 2
NB0 = 80
NB1 = 80
TB = NS * (NB0 + NB1)


def _sc_body(with_deg, tbm, h_hbm, e3_hbm, pad3_hbm, *rest):
    if with_deg:
        (acc_out, deg_out, ebuf, rows_a, rows_b, ones_v,
         acc_sh, deg_sh, semA0, semA1, semB0, semB1, semD) = rest
    else:
        (acc_out, ebuf, rows_a, rows_b,
         acc_sh, semA0, semA1, semB0, semB1) = rest
    mix = tbm % CH
    c = lax.axis_index("c")
    s = lax.axis_index("s")

    zeros16 = jnp.zeros((16,), jnp.float32)
    ones16 = jnp.ones((16,), jnp.float32)

    def _zero_row_body(i, carry):
        for k in range(D // 16):
            rows_a[i, pl.ds(k * 16, 16)] = zeros16
        return carry

    lax.fori_loop(0, BATCH, _zero_row_body, 0)
    if with_deg:
        for k in range(BATCH // 16):
            ones_v[pl.ds(k * 16, 16)] = ones16

    base = s * ACC_PER_TILE
    n_full = ACC_PER_TILE // BATCH
    rem = ACC_PER_TILE - n_full * BATCH
    zcopies = []
    for k in range(n_full):
        zcopies.append(pltpu.async_copy(
            rows_a, acc_sh.at[pl.ds(base + k * BATCH, BATCH)], semB0))
    if rem:
        zcopies.append(pltpu.async_copy(
            rows_a.at[pl.ds(0, rem)],
            acc_sh.at[pl.ds(base + n_full * BATCH, rem)], semB0))
    if with_deg:
        dbase = s * DEG_PER_TILE
        for k in range(DEG_PER_TILE // BATCH):
            zcopies.append(pltpu.async_copy(
                rows_a.at[0], deg_sh.at[pl.ds(dbase + k * BATCH, BATCH)], semB0))
    for cp in zcopies:
        cp.wait()
    plsc.subcore_barrier()

    nchunks = jnp.where(c == 0, NB0 // CH, NB1 // CH)
    base_b = jnp.where(c == 0, s * NB0, NS * NB0 + s * NB1)

    def _chunk_body(kc, carry):
        c0 = base_b + kc * CH

        @pl.when(c0 + CH <= tbm)
        def _():
            pltpu.sync_copy(e3_hbm.at[pl.ds(c0, CH)], ebuf)

        if mix:
            @pl.when(c0 == tbm - mix)
            def _():
                pltpu.sync_copy(e3_hbm.at[pl.ds(tbm - mix, mix)],
                                ebuf.at[pl.ds(0, mix)])
                pltpu.sync_copy(pad3_hbm.at[pl.ds(0, CH - mix)],
                                ebuf.at[pl.ds(mix, CH - mix)])

        @pl.when(c0 >= tbm + (CH - mix if mix else 0))
        def _():
            pltpu.sync_copy(pad3_hbm.at[pl.ds(c0 - tbm, CH)], ebuf)

        pltpu.async_copy(h_hbm.at[ebuf.at[0].at[0]], rows_a, semA0)

        def _pair_body(p, carry):
            j0 = 2 * p
            j1 = j0 + 1

            @pl.when(p > 0)
            def _():
                pltpu.make_async_copy(
                    rows_b, acc_sh.at[ebuf.at[j0 - 1].at[1]], semB1).wait()

            pltpu.async_copy(h_hbm.at[ebuf.at[j1].at[0]], rows_b, semA1)
            pltpu.make_async_copy(h_hbm.at[ebuf.at[j0].at[0]], rows_a, semA0).wait()
            pltpu.async_copy(rows_a, acc_sh.at[ebuf.at[j0].at[1]], semB0, add=True)
            if with_deg:
                pltpu.async_copy(ones_v, deg_sh.at[ebuf.at[j0].at[1]], semD, add=True)
            pltpu.make_async_copy(h_hbm.at[ebuf.at[j1].at[0]], rows_b, semA1).wait()
            pltpu.make_async_copy(rows_a, acc_sh.at[ebuf.at[j0].at[1]], semB0).wait()

            @pl.when(p + 1 < NPAIR)
            def _():
                pltpu.async_copy(h_hbm.at[ebuf.at[j0 + 2].at[0]], rows_a, semA0)

            pltpu.async_copy(rows_b, acc_sh.at[ebuf.at[j1].at[1]], semB1, add=True)
            if with_deg:
                pltpu.async_copy(ones_v, deg_sh.at[ebuf.at[j1].at[1]], semD, add=True)
            return carry

        lax.fori_loop(0, NPAIR, _pair_body, 0)
        pltpu.make_async_copy(rows_b, acc_sh.at[ebuf.at[CH - 1].at[1]], semB1).wait()
        if with_deg:
            def _deg_drain(p, carry2):
                pltpu.make_async_copy(ones_v, deg_sh.at[ebuf.at[0].at[1]], semD).wait()
                return carry2

            lax.fori_loop(0, CH, _deg_drain, 0)
        return carry

    lax.fori_loop(0, nchunks, _chunk_body, 0)
    plsc.subcore_barrier()

    pltpu.sync_copy(acc_sh.at[pl.ds(base, ACC_PER_TILE)],
                    acc_out.at[c].at[pl.ds(base, ACC_PER_TILE)])
    if with_deg:
        pltpu.sync_copy(deg_sh.at[pl.ds(dbase, DEG_PER_TILE)],
                        deg_out.at[c].at[pl.ds(dbase, DEG_PER_TILE)])


def _make_sc_segment_sum(with_deg, tbm):
    out_type = [jax.ShapeDtypeStruct((NC, ACC_ROWS, D), jnp.float32)]
    scratch = [
        pltpu.VMEM((CH, 2, BATCH), jnp.int32),
        pltpu.VMEM((BATCH, D), jnp.float32),
        pltpu.VMEM((BATCH, D), jnp.float32),
    ]
    if with_deg:
        out_type.append(jax.ShapeDtypeStruct((NC, DEG_ROWS), jnp.float32))
        scratch.append(pltpu.VMEM((BATCH,), jnp.float32))
    scratch.append(pltpu.VMEM_SHARED((ACC_ROWS, D), jnp.float32))
    if with_deg:
        scratch.append(pltpu.VMEM_SHARED((DEG_ROWS,), jnp.float32))
    scratch += [pltpu.SemaphoreType.DMA] * (5 if with_deg else 4)
    return functools.partial(
        pl.kernel,
        mesh=_mesh,
        out_type=out_type,
        scratch_types=scratch,
    )(functools.partial(_sc_body, with_deg, tbm))


def _dense_body(accA_ref, accB_ref, degs_ref, h_ref,
                Wl_ref, Wr_ref, b_ref, z_ref, stats_ref):
    i = pl.program_id(0)
    deg = degs_ref[:, 0:1] + degs_ref[:, 1:2]
    inv = 1.0 / jnp.maximum(deg, 1.0)
    S = (accA_ref[0] + accB_ref[0]) * inv
    z = lax.dot_general(S, Wl_ref[...], (((1,), (1,)), ((), ())),
                        preferred_element_type=jnp.float32)
    z = z + lax.dot_general(h_ref[...], Wr_ref[...], (((1,), (1,)), ((), ())),
                            preferred_element_type=jnp.float32)
    z = z + b_ref[...]
    z_ref[...] = z
    s0 = jnp.sum(z, axis=0, keepdims=True)
    s1 = jnp.sum(z * z, axis=0, keepdims=True)
    upd = jnp.concatenate([s0, s1, jnp.zeros((6, D), jnp.float32)], axis=0)

    @pl.when(i == 0)
    def _():
        stats_ref[...] = jnp.zeros((8, D), jnp.float32)

    stats_ref[...] += upd


def _bn_body(z_ref, stats_ref, gamma_ref, beta_ref, out_ref):
    st = stats_ref[...]
    mu = st[0:1, :] * (1.0 / N)
    var = st[1:2, :] * (1.0 / N) - mu * mu
    a = gamma_ref[...] * lax.rsqrt(var + EPS)
    out_ref[...] = jnp.maximum((z_ref[...] - mu) * a + beta_ref[...], 0.0)


def _dense_bn_relu(acc3d, degs, h, Wl, Wr, b, gamma, beta):
    R = 2000
    G = N // R
    row = lambda i: (i, 0)
    const = lambda i: (0, 0)
    z, stats = pl.pallas_call(
        _dense_body,
        grid=(G,),
        in_specs=[
            pl.BlockSpec((1, R, D), lambda i: (0, i, 0)),
            pl.BlockSpec((1, R, D), lambda i: (1, i, 0)),
            pl.BlockSpec((R, NC), lambda i: (i, 0)),
            pl.BlockSpec((R, D), row),
            pl.BlockSpec((D, D), const),
            pl.BlockSpec((D, D), const),
            pl.BlockSpec((1, D), const),
        ],
        out_specs=[
            pl.BlockSpec((R, D), row),
            pl.BlockSpec((8, D), const),
        ],
        out_shape=[
            jax.ShapeDtypeStruct((N, D), jnp.float32),
            jax.ShapeDtypeStruct((8, D), jnp.float32),
        ],
    )(acc3d, acc3d, degs, h, Wl, Wr, b)
    return pl.pallas_call(
        _bn_body,
        grid=(G,),
        in_specs=[
            pl.BlockSpec((R, D), row),
            pl.BlockSpec((8, D), const),
            pl.BlockSpec((1, D), const),
            pl.BlockSpec((1, D), const),
        ],
        out_specs=pl.BlockSpec((R, D), row),
        out_shape=jax.ShapeDtypeStruct((N, D), jnp.float32),
    )(z, stats, gamma, beta)


def kernel(x, edge_index, W1l, W1r, b1, gamma1, beta1, W2l, W2r, b2, gamma2, beta2):
    E = edge_index.shape[1]
    tbm = E // BATCH
    npad = TB * BATCH - E
    e3 = jnp.transpose(edge_index.reshape(2, tbm, BATCH), (1, 0, 2))
    pad_src = (np.arange(npad, dtype=np.int64) * 97 % N).astype(np.int32)
    pad_dst = (N + np.arange(npad, dtype=np.int64) % (ACC_ROWS - N)).astype(np.int32)
    pad3 = jnp.asarray(np.stack(
        [pad_src.reshape(-1, BATCH), pad_dst.reshape(-1, BATCH)], axis=1))

    seg_deg = _make_sc_segment_sum(True, tbm)
    seg = _make_sc_segment_sum(False, tbm)

    def dense(acc, deg, h, Wl, Wr, b, gamma, beta):
        return _dense_bn_relu(
            acc, jnp.transpose(deg),
            h, Wl, Wr, b.reshape(1, D), gamma.reshape(1, D), beta.reshape(1, D))

    acc1, deg = seg_deg(x, e3, pad3)
    h1 = dense(acc1, deg, x, W1l, W1r, b1, gamma1, beta1)
    acc2, = seg(h1, e3, pad3)
    return dense(acc2, deg, h1, W2l, W2r, b2, gamma2, beta2)
